# Initial kernel scaffold; baseline (speedup 1.0000x reference)
import functools

import numpy as np
import jax
import jax.numpy as jnp
from jax import lax
from jax.experimental import pallas as pl
from jax.experimental.pallas import tpu as pltpu
from jax.experimental.pallas import tpu_sc as plsc

N = 10000
E = 320000
D = 128
DE = 16
HID = 128
OUT = 128
G = 64
HEADS = 4
NT = 1

NC = 2
NS = 16
NW = NC * NS
EPW = E // NW
C = 80
NCHUNK = EPW // C
RPS = N // NS
ZR = 125


_sc_mesh = plsc.VectorSubcoreMesh(
    core_axis_name="c", subcore_axis_name="s", num_cores=NC, num_subcores=NS)


@functools.partial(
    pl.kernel,
    out_type=jax.ShapeDtypeStruct((NC * N, OUT), jnp.float32),
    mesh=_sc_mesh,
    scratch_types=[
        pltpu.VMEM((C,), jnp.int32),
        pltpu.VMEM((C,), jnp.int32),
        pltpu.VMEM((C, OUT), jnp.float32),
        pltpu.VMEM((C, OUT), jnp.float32),
        pltpu.VMEM((ZR, OUT), jnp.float32),
        pltpu.VMEM_SHARED((N, OUT), jnp.float32),
        pltpu.SemaphoreType.DMA,
    ],
)
def _sc_msgpass(h_hbm, e_hbm, src_hbm, dst_hbm, out_hbm,
                src_v, dst_v, hsrc_v, eemb_v, zbuf_v, agg_sh, sem):
  c = lax.axis_index("c")
  s = lax.axis_index("s")
  wid = c * NS + s

  def zrow(r, carry):
    for j in range(OUT // 16):
      zbuf_v[r, pl.ds(j * 16, 16)] = jnp.zeros((16,), jnp.float32)
    return carry

  lax.fori_loop(0, ZR, zrow, 0)

  def zslice(i, carry):
    pltpu.sync_copy(zbuf_v, agg_sh.at[pl.ds(s * RPS + i * ZR, ZR)])
    return carry

  lax.fori_loop(0, RPS // ZR, zslice, 0)
  plsc.subcore_barrier()

  base = wid * EPW

  def chunk(i, carry):
    off = base + i * C
    pltpu.sync_copy(src_hbm.at[pl.ds(off, C)], src_v)
    pltpu.sync_copy(dst_hbm.at[pl.ds(off, C)], dst_v)
    pltpu.async_copy(h_hbm.at[src_v], hsrc_v, sem).wait()
    pltpu.sync_copy(e_hbm.at[pl.ds(off, C)], eemb_v)

    def row(r, rc):
      for j in range(OUT // 16):
        sl = pl.ds(j * 16, 16)
        hsrc_v[r, sl] = jnp.maximum(hsrc_v[r, sl] + eemb_v[r, sl], 0.0)
      return rc

    lax.fori_loop(0, C, row, 0)
    pltpu.sync_copy(hsrc_v, agg_sh.at[dst_v], add=True)
    return carry

  lax.fori_loop(0, NCHUNK, chunk, 0)
  plsc.subcore_barrier()
  pltpu.sync_copy(agg_sh.at[pl.ds(s * RPS, RPS)],
                  out_hbm.at[pl.ds(c * N + s * RPS, RPS)])


def _sc_layer(h, e_emb, src, dst):
  out = _sc_msgpass(h, e_emb, src, dst)
  return out.reshape(NC, N, OUT)


_BE = 4000


def _eemb_body(attr_ref, w_ref, b_ref, o_ref):
  o_ref[...] = (
      jnp.dot(attr_ref[...], w_ref[...], preferred_element_type=jnp.float32)
      + b_ref[...])


def _eemb_call(edge_attr, We, be):
  return pl.pallas_call(
      _eemb_body,
      grid=(E // _BE,),
      in_specs=[
          pl.BlockSpec((_BE, DE), lambda e: (e, 0)),
          pl.BlockSpec((DE, OUT), lambda e: (0, 0)),
          pl.BlockSpec((1, OUT), lambda e: (0, 0)),
      ],
      out_specs=pl.BlockSpec((_BE, OUT), lambda e: (e, 0)),
      out_shape=jax.ShapeDtypeStruct((E, OUT), jnp.float32),
  )(edge_attr, We, be.reshape(1, OUT))


def _mlp_body(h_ref, a_ref, w1_ref, b1_ref, w2_ref, b2_ref, g_ref, b_ref,
              o_ref):
  h2 = h_ref[...] + a_ref[0] + a_ref[1]
  t = jnp.maximum(
      jnp.dot(h2, w1_ref[...], preferred_element_type=jnp.float32)
      + b1_ref[...], 0.0)
  h3 = (jnp.dot(t, w2_ref[...], preferred_element_type=jnp.float32)
        + b2_ref[...])
  mu = jnp.mean(h3, axis=0, keepdims=True)
  var = jnp.mean((h3 - mu) ** 2, axis=0, keepdims=True)
  o_ref[...] = jnp.maximum(
      (h3 - mu) * lax.rsqrt(var + 1e-5) * g_ref[...] + b_ref[...], 0.0)


def _mlp_call(h, agg2, w1, b1, w2, b2, bng, bnb):
  return pl.pallas_call(
      _mlp_body,
      out_shape=jax.ShapeDtypeStruct((N, HID), jnp.float32),
  )(h, agg2, w1, b1.reshape(1, -1), w2, b2.reshape(1, -1),
    bng.reshape(1, -1), bnb.reshape(1, -1))


def _pool_body(h_ref, batch_ref, gw_ref, gb_ref, o_ref):
  h = h_ref[...]
  gate = (jnp.dot(h, gw_ref[...], preferred_element_type=jnp.float32)
          + gb_ref[...])
  seg = batch_ref[...]
  ids = lax.broadcasted_iota(jnp.int32, (N, G), 1)
  onehot = seg == ids
  neg = jnp.float32(-jnp.inf)
  masked = jnp.where(onehot, gate, neg)
  m = jnp.max(masked, axis=0, keepdims=True)
  m = jnp.where(m > neg, m, 0.0)
  oh = onehot.astype(jnp.float32)
  m_n = jnp.sum(oh * m, axis=1, keepdims=True)
  ex = jnp.exp(gate - m_n)
  s_row = jnp.sum(oh * ex, axis=0, keepdims=True)
  s_n = jnp.sum(oh * s_row, axis=1, keepdims=True)
  alpha = ex / (s_n + 1e-16)
  o_ref[...] = lax.dot_general(
      oh, alpha * h, (((0,), (0,)), ((), ())),
      preferred_element_type=jnp.float32)


def _pool_call(h, batch, gw, gb):
  return pl.pallas_call(
      _pool_body,
      out_shape=jax.ShapeDtypeStruct((G, HID), jnp.float32),
  )(h, batch.reshape(N, 1), gw, gb.reshape(1, 1))


_NN = G * 7
_P = np.zeros((_NN, _NN), np.float32)
_P[np.arange(_NN), (np.arange(_NN) // 7) * 7] = 1.0
_S = np.zeros((HEADS * HID, HEADS), np.float32)
_S[np.arange(HEADS * HID), np.arange(HEADS * HID) // HID] = 1.0
_St = _S.T.copy()
_Hm = np.zeros((HEADS * HID, HID), np.float32)
_Hm[np.arange(HEADS * HID), np.arange(HEADS * HID) % HID] = 1.0 / HEADS
_CMASK = (np.arange(_NN) % 7 != 0).astype(np.float32).reshape(_NN, 1)
_CSEL = np.zeros((G, _NN), np.float32)
_CSEL[np.arange(G), np.arange(G) * 7] = 1.0


def _gat_body(nodes_ref, wl_ref, bl_ref, wr_ref, br_ref, att_ref, bias_ref,
              g_ref, b_ref, f1w_ref, f1b_ref, f2w_ref, f2b_ref,
              p_ref, s_ref, st_ref, hm_ref, c_ref, csel_ref, o_ref):
  nodes = nodes_ref[...]
  xl = (jnp.dot(nodes, wl_ref[...], preferred_element_type=jnp.float32)
        + bl_ref[...])
  xr = (jnp.dot(nodes, wr_ref[...], preferred_element_type=jnp.float32)
        + br_ref[...])
  xlc = jnp.dot(p_ref[...], xl, preferred_element_type=jnp.float32)
  att = att_ref[...]

  def leaky(v):
    return jnp.where(v >= 0, v, 0.2 * v)

  es = leaky(xl + xr)
  ec = leaky(xlc + xr)
  ls = jnp.dot(es * att, s_ref[...], preferred_element_type=jnp.float32)
  lc = jnp.dot(ec * att, s_ref[...], preferred_element_type=jnp.float32)
  cm = c_ref[...]
  neg = jnp.float32(-jnp.inf)
  m = jnp.maximum(ls, jnp.where(cm > 0, lc, neg))
  a_s = jnp.exp(ls - m)
  a_c = jnp.where(cm > 0, jnp.exp(lc - m), 0.0)
  den = a_s + a_c + 1e-16
  w_s = jnp.dot(a_s / den, st_ref[...], preferred_element_type=jnp.float32)
  w_c = jnp.dot(a_c / den, st_ref[...], preferred_element_type=jnp.float32)
  out512 = w_s * xl + w_c * xlc
  out = (jnp.dot(out512, hm_ref[...], preferred_element_type=jnp.float32)
         + bias_ref[...])
  mu = jnp.mean(out, axis=0, keepdims=True)
  var = jnp.mean((out - mu) ** 2, axis=0, keepdims=True)
  out = jnp.maximum(
      (out - mu) * lax.rsqrt(var + 1e-5) * g_ref[...] + b_ref[...], 0.0)
  cen = jnp.dot(csel_ref[...], out, preferred_element_type=jnp.float32)
  hfc = jnp.maximum(
      jnp.dot(cen, f1w_ref[...], preferred_element_type=jnp.float32)
      + f1b_ref[...], 0.0)
  o_ref[...] = (jnp.dot(hfc, f2w_ref[...], preferred_element_type=jnp.float32)
                + f2b_ref[...])


def _gat_call(nodes, p):
  return pl.pallas_call(
      _gat_body,
      out_shape=jax.ShapeDtypeStruct((G, NT), jnp.float32),
  )(nodes, p['gat_Wl'], p['gat_bl'].reshape(1, -1),
    p['gat_Wr'], p['gat_br'].reshape(1, -1),
    p['gat_att'].reshape(1, HEADS * HID), p['gat_bias'].reshape(1, -1),
    p['nbn_g'].reshape(1, -1), p['nbn_b'].reshape(1, -1),
    p['fc1_W'], p['fc1_b'].reshape(1, -1),
    p['fc2_W'], p['fc2_b'].reshape(1, -1),
    jnp.asarray(_P), jnp.asarray(_S), jnp.asarray(_St), jnp.asarray(_Hm),
    jnp.asarray(_CMASK), jnp.asarray(_CSEL))


def kernel(x, edge_index, edge_attr, batch, ECFP, Topological, MACCS, EState,
           Rdkit2D, Phar2D, params):
  p = params
  src = edge_index[0]
  dst = edge_index[1]
  h = x
  for l in range(4):
    e_emb = _eemb_call(edge_attr, p['g%d_We' % l], p['g%d_be' % l])
    agg2 = _sc_layer(h, e_emb, src, dst)
    h = _mlp_call(h, agg2, p['g%d_W1' % l], p['g%d_b1' % l],
                  p['g%d_W2' % l], p['g%d_b2' % l],
                  p['bn%d_g' % l], p['bn%d_b' % l])
  go = _pool_call(h, batch, p['gate_W'], p['gate_b'])
  nodes = jnp.stack(
      [go, ECFP, Topological, MACCS, EState, Rdkit2D, Phar2D],
      axis=1).reshape(_NN, OUT)
  return _gat_call(nodes, p)


# speedup vs baseline: 2.5472x; 2.5472x over previous
import functools

import numpy as np
import jax
import jax.numpy as jnp
from jax import lax
from jax.experimental import pallas as pl
from jax.experimental.pallas import tpu as pltpu
from jax.experimental.pallas import tpu_sc as plsc

N = 10000
E = 320000
D = 128
DE = 16
HID = 128
OUT = 128
G = 64
HEADS = 4
NT = 1

NC = 2
NS = 16
NW = NC * NS
EPW = E // NW
C = 80
NCHUNK = EPW // C
NPAD = 10240
RPS = NPAD // NS
ZR = 128


def _sc_msgpass_body(h_hbm, e_hbm, src_hbm, dst_hbm, out_hbm,
                     src_v, dst_v, hsrc_v, eemb_v, zbuf_v, agg_sh, sem):
  c = lax.axis_index("c")
  s = lax.axis_index("s")
  wid = c * NS + s

  def zrow(r, carry):
    for j in range(OUT // 16):
      zbuf_v[r, pl.ds(j * 16, 16)] = jnp.zeros((16,), jnp.float32)
    return carry

  lax.fori_loop(0, ZR, zrow, 0)

  def zslice(i, carry):
    pltpu.sync_copy(zbuf_v, agg_sh.at[pl.ds(s * RPS + i * ZR, ZR)])
    return carry

  lax.fori_loop(0, RPS // ZR, zslice, 0)
  plsc.subcore_barrier()

  base = wid * EPW

  def chunk(i, carry):
    off = base + i * C
    pltpu.sync_copy(src_hbm.at[pl.ds(off, C)], src_v)
    pltpu.sync_copy(dst_hbm.at[pl.ds(off, C)], dst_v)
    pltpu.async_copy(h_hbm.at[src_v], hsrc_v, sem).wait()
    pltpu.sync_copy(e_hbm.at[pl.ds(off, C)], eemb_v)

    def row(r, rc):
      for j in range(OUT // 16):
        sl = pl.ds(j * 16, 16)
        hsrc_v[r, sl] = jnp.maximum(hsrc_v[r, sl] + eemb_v[r, sl], 0.0)
      return rc

    lax.fori_loop(0, C, row, 0)
    pltpu.sync_copy(hsrc_v, agg_sh.at[dst_v], add=True)
    return carry

  lax.fori_loop(0, NCHUNK, chunk, 0)
  plsc.subcore_barrier()
  pltpu.sync_copy(agg_sh.at[pl.ds(s * RPS, RPS)],
                  out_hbm.at[pl.ds(c * NPAD + s * RPS, RPS)])


@functools.cache
def _get_sc_msgpass():
  mesh = plsc.VectorSubcoreMesh(
      core_axis_name="c", subcore_axis_name="s",
      num_cores=NC, num_subcores=NS)
  return pl.kernel(
      _sc_msgpass_body,
      out_type=jax.ShapeDtypeStruct((NC * NPAD, OUT), jnp.float32),
      mesh=mesh,
      scratch_types=[
          pltpu.VMEM((C,), jnp.int32),
          pltpu.VMEM((C,), jnp.int32),
          pltpu.VMEM((C, OUT), jnp.float32),
          pltpu.VMEM((C, OUT), jnp.float32),
          pltpu.VMEM((ZR, OUT), jnp.float32),
          pltpu.VMEM_SHARED((NPAD, OUT), jnp.float32),
          pltpu.SemaphoreType.DMA,
      ],
  )


def _sc_layer(h, e_emb, src, dst):
  out = _get_sc_msgpass()(h, e_emb, src, dst)
  return out.reshape(NC, NPAD, OUT)


_BE = 4000


def _eemb_body(attr_ref, w_ref, b_ref, o_ref):
  o_ref[...] = (
      jnp.dot(attr_ref[...], w_ref[...], preferred_element_type=jnp.float32,
              precision=lax.Precision.HIGHEST)
      + b_ref[...])


def _eemb_call(edge_attr, We, be):
  return pl.pallas_call(
      _eemb_body,
      grid=(E // _BE,),
      in_specs=[
          pl.BlockSpec((_BE, DE), lambda e: (e, 0)),
          pl.BlockSpec((DE, OUT), lambda e: (0, 0)),
          pl.BlockSpec((1, OUT), lambda e: (0, 0)),
      ],
      out_specs=pl.BlockSpec((_BE, OUT), lambda e: (e, 0)),
      out_shape=jax.ShapeDtypeStruct((E, OUT), jnp.float32),
  )(edge_attr, We, be.reshape(1, OUT))


def _mlp_body(h_ref, a_ref, w1_ref, b1_ref, w2_ref, b2_ref, g_ref, b_ref,
              o_ref):
  h2 = h_ref[...] + a_ref[0, :N] + a_ref[1, :N]
  t = jnp.maximum(
      jnp.dot(h2, w1_ref[...], preferred_element_type=jnp.float32,
              precision=lax.Precision.HIGHEST)
      + b1_ref[...], 0.0)
  h3 = (jnp.dot(t, w2_ref[...], preferred_element_type=jnp.float32,
              precision=lax.Precision.HIGHEST)
        + b2_ref[...])
  mu = jnp.mean(h3, axis=0, keepdims=True)
  var = jnp.mean((h3 - mu) ** 2, axis=0, keepdims=True)
  o_ref[...] = jnp.maximum(
      (h3 - mu) * lax.rsqrt(var + 1e-5) * g_ref[...] + b_ref[...], 0.0)


def _mlp_call(h, agg2, w1, b1, w2, b2, bng, bnb):
  return pl.pallas_call(
      _mlp_body,
      out_shape=jax.ShapeDtypeStruct((N, HID), jnp.float32),
  )(h, agg2, w1, b1.reshape(1, -1), w2, b2.reshape(1, -1),
    bng.reshape(1, -1), bnb.reshape(1, -1))


def _pool_body(h_ref, batch_ref, gw_ref, gb_ref, o_ref):
  h = h_ref[...]
  gate = (jnp.dot(h, gw_ref[...], preferred_element_type=jnp.float32,
              precision=lax.Precision.HIGHEST)
          + gb_ref[...])
  seg = batch_ref[...]
  ids = lax.broadcasted_iota(jnp.int32, (N, G), 1)
  onehot = seg == ids
  neg = jnp.float32(-jnp.inf)
  masked = jnp.where(onehot, gate, neg)
  m = jnp.max(masked, axis=0, keepdims=True)
  m = jnp.where(m > neg, m, 0.0)
  oh = onehot.astype(jnp.float32)
  m_n = jnp.sum(oh * m, axis=1, keepdims=True)
  ex = jnp.exp(gate - m_n)
  s_row = jnp.sum(oh * ex, axis=0, keepdims=True)
  s_n = jnp.sum(oh * s_row, axis=1, keepdims=True)
  alpha = ex / (s_n + 1e-16)
  o_ref[...] = lax.dot_general(
      oh, alpha * h, (((0,), (0,)), ((), ())),
      preferred_element_type=jnp.float32,
              precision=lax.Precision.HIGHEST)


def _pool_call(h, batch, gw, gb):
  return pl.pallas_call(
      _pool_body,
      out_shape=jax.ShapeDtypeStruct((G, HID), jnp.float32),
  )(h, batch.reshape(N, 1), gw, gb.reshape(1, 1))


_NN = G * 7
_P = np.zeros((_NN, _NN), np.float32)
_P[np.arange(_NN), (np.arange(_NN) // 7) * 7] = 1.0
_S = np.zeros((HEADS * HID, HEADS), np.float32)
_S[np.arange(HEADS * HID), np.arange(HEADS * HID) // HID] = 1.0
_St = _S.T.copy()
_Hm = np.zeros((HEADS * HID, HID), np.float32)
_Hm[np.arange(HEADS * HID), np.arange(HEADS * HID) % HID] = 1.0 / HEADS
_CMASK = (np.arange(_NN) % 7 != 0).astype(np.float32).reshape(_NN, 1)
_CSEL = np.zeros((G, _NN), np.float32)
_CSEL[np.arange(G), np.arange(G) * 7] = 1.0


def _gat_body(nodes_ref, wl_ref, bl_ref, wr_ref, br_ref, att_ref, bias_ref,
              g_ref, b_ref, f1w_ref, f1b_ref, f2w_ref, f2b_ref,
              p_ref, s_ref, st_ref, hm_ref, c_ref, csel_ref, o_ref):
  nodes = nodes_ref[...]
  xl = (jnp.dot(nodes, wl_ref[...], preferred_element_type=jnp.float32,
              precision=lax.Precision.HIGHEST)
        + bl_ref[...])
  xr = (jnp.dot(nodes, wr_ref[...], preferred_element_type=jnp.float32,
              precision=lax.Precision.HIGHEST)
        + br_ref[...])
  xlc = jnp.dot(p_ref[...], xl, preferred_element_type=jnp.float32,
              precision=lax.Precision.HIGHEST)
  att = att_ref[...]

  def leaky(v):
    return jnp.where(v >= 0, v, 0.2 * v)

  es = leaky(xl + xr)
  ec = leaky(xlc + xr)
  ls = jnp.dot(es * att, s_ref[...], preferred_element_type=jnp.float32,
              precision=lax.Precision.HIGHEST)
  lc = jnp.dot(ec * att, s_ref[...], preferred_element_type=jnp.float32,
              precision=lax.Precision.HIGHEST)
  cm = c_ref[...]
  neg = jnp.float32(-jnp.inf)
  m = jnp.maximum(ls, jnp.where(cm > 0, lc, neg))
  a_s = jnp.exp(ls - m)
  a_c = jnp.where(cm > 0, jnp.exp(lc - m), 0.0)
  den = a_s + a_c + 1e-16
  w_s = jnp.dot(a_s / den, st_ref[...], preferred_element_type=jnp.float32,
              precision=lax.Precision.HIGHEST)
  w_c = jnp.dot(a_c / den, st_ref[...], preferred_element_type=jnp.float32,
              precision=lax.Precision.HIGHEST)
  out512 = w_s * xl + w_c * xlc
  out = (jnp.dot(out512, hm_ref[...], preferred_element_type=jnp.float32,
              precision=lax.Precision.HIGHEST)
         + bias_ref[...])
  mu = jnp.mean(out, axis=0, keepdims=True)
  var = jnp.mean((out - mu) ** 2, axis=0, keepdims=True)
  out = jnp.maximum(
      (out - mu) * lax.rsqrt(var + 1e-5) * g_ref[...] + b_ref[...], 0.0)
  cen = jnp.dot(csel_ref[...], out, preferred_element_type=jnp.float32,
              precision=lax.Precision.HIGHEST)
  hfc = jnp.maximum(
      jnp.dot(cen, f1w_ref[...], preferred_element_type=jnp.float32,
              precision=lax.Precision.HIGHEST)
      + f1b_ref[...], 0.0)
  o_ref[...] = (jnp.dot(hfc, f2w_ref[...], preferred_element_type=jnp.float32,
              precision=lax.Precision.HIGHEST)
                + f2b_ref[...])


def _gat_call(nodes, p):
  return pl.pallas_call(
      _gat_body,
      out_shape=jax.ShapeDtypeStruct((G, NT), jnp.float32),
  )(nodes, p['gat_Wl'], p['gat_bl'].reshape(1, -1),
    p['gat_Wr'], p['gat_br'].reshape(1, -1),
    p['gat_att'].reshape(1, HEADS * HID), p['gat_bias'].reshape(1, -1),
    p['nbn_g'].reshape(1, -1), p['nbn_b'].reshape(1, -1),
    p['fc1_W'], p['fc1_b'].reshape(1, -1),
    p['fc2_W'], p['fc2_b'].reshape(1, -1),
    jnp.asarray(_P), jnp.asarray(_S), jnp.asarray(_St), jnp.asarray(_Hm),
    jnp.asarray(_CMASK), jnp.asarray(_CSEL))


def kernel(x, edge_index, edge_attr, batch, ECFP, Topological, MACCS, EState,
           Rdkit2D, Phar2D, params):
  p = params
  src = edge_index[0]
  dst = edge_index[1]
  h = x
  for l in range(4):
    e_emb = _eemb_call(edge_attr, p['g%d_We' % l], p['g%d_be' % l])
    agg2 = _sc_layer(h, e_emb, src, dst)
    h = _mlp_call(h, agg2, p['g%d_W1' % l], p['g%d_b1' % l],
                  p['g%d_W2' % l], p['g%d_b2' % l],
                  p['bn%d_g' % l], p['bn%d_b' % l])
  go = _pool_call(h, batch, p['gate_W'], p['gate_b'])
  nodes = jnp.stack(
      [go, ECFP, Topological, MACCS, EState, Rdkit2D, Phar2D],
      axis=1).reshape(_NN, OUT)
  return _gat_call(nodes, p)
